# Initial kernel scaffold; baseline (speedup 1.0000x reference)
import jax
import jax.numpy as jnp
from jax.experimental import pallas as pl


def kernel(x, table, W1, b1, W2, b2):
    raise NotImplementedError("write your pallas kernel here")


# speedup vs baseline: 2.8781x; 2.8781x over previous
import functools

import jax
import jax.numpy as jnp
from jax import lax
from jax.experimental import pallas as pl
from jax.experimental.pallas import tpu as pltpu
from jax.experimental.pallas import tpu_sc as plsc

EMB = 64
HID = 256
OUT = 48
B = 16384
WIN = 5

NW = 32
ROWS = B * WIN
ROWS_PER_W = ROWS // NW
CHUNK = 128
NCHUNK = ROWS_PER_W // CHUNK

@functools.cache
def _build_sc_gather():
    mesh = plsc.VectorSubcoreMesh(core_axis_name="c", subcore_axis_name="s")

    @functools.partial(
        pl.kernel,
        out_type=jax.ShapeDtypeStruct((ROWS, EMB), jnp.float32),
        mesh=mesh,
        scratch_types=[
            pltpu.VMEM((NCHUNK, CHUNK), jnp.int32),
            pltpu.VMEM((2, CHUNK, EMB), jnp.float32),
            pltpu.SemaphoreType.DMA,
            pltpu.SemaphoreType.DMA,
            pltpu.SemaphoreType.DMA,
        ],
        compiler_params=pltpu.CompilerParams(use_tc_tiling_on_sc=False),
    )
    def _sc_gather(x_hbm, table_hbm, out_hbm, idx_v, rows_v, gsem, osem0, osem1):
        wid = lax.axis_index("s") * 2 + lax.axis_index("c")
        pltpu.sync_copy(x_hbm.at[wid], idx_v)
        osems = (osem0, osem1)
        base = wid * ROWS_PER_W
        for j in range(NCHUNK):
            slot = j % 2
            buf = rows_v.at[slot]
            gather = pltpu.async_copy(table_hbm.at[idx_v.at[j]], buf, gsem)
            if j >= 2:
                pltpu.make_async_copy(
                    rows_v.at[slot],
                    out_hbm.at[pl.ds(base + (j - 2) * CHUNK, CHUNK)],
                    osems[slot],
                ).wait()
            gather.wait()
            pltpu.async_copy(
                buf, out_hbm.at[pl.ds(base + j * CHUNK, CHUNK)], osems[slot]
            )
        for j in (NCHUNK - 2, NCHUNK - 1):
            slot = j % 2
            pltpu.make_async_copy(
                rows_v.at[slot], out_hbm.at[pl.ds(base + j * CHUNK, CHUNK)],
                osems[slot],
            ).wait()

    return _sc_gather


def _mlp_body(flat_ref, w1_ref, b1_ref, w2_ref, b2_ref, out_ref):
    h = jnp.tanh(
        jnp.dot(flat_ref[...], w1_ref[...], preferred_element_type=jnp.float32)
        + b1_ref[...]
    )
    out_ref[...] = (
        jnp.dot(h, w2_ref[...], preferred_element_type=jnp.float32) + b2_ref[...]
    )


BM = 2048


@jax.jit
def kernel(x, table, W1, b1, W2, b2):
    x_r = x.reshape(NW, NCHUNK, CHUNK)
    flat = _build_sc_gather()(x_r, table)
    flat = flat.reshape(B, WIN * EMB)
    out = pl.pallas_call(
        _mlp_body,
        grid=(B // BM,),
        in_specs=[
            pl.BlockSpec((BM, WIN * EMB), lambda i: (i, 0)),
            pl.BlockSpec((WIN * EMB, HID), lambda i: (0, 0)),
            pl.BlockSpec((1, HID), lambda i: (0, 0)),
            pl.BlockSpec((HID, OUT), lambda i: (0, 0)),
            pl.BlockSpec((1, OUT), lambda i: (0, 0)),
        ],
        out_specs=pl.BlockSpec((BM, OUT), lambda i: (i, 0)),
        out_shape=jax.ShapeDtypeStruct((B, OUT), jnp.float32),
    )(flat, W1, b1.reshape(1, HID), W2, b2.reshape(1, OUT))
    return out
